# Initial kernel scaffold; baseline (speedup 1.0000x reference)
import jax
import jax.numpy as jnp
from jax.experimental import pallas as pl


def kernel(inputs, rel_emb_v_weight):
    raise NotImplementedError("write your pallas kernel here")


# speedup vs baseline: 1.6808x; 1.6808x over previous
import functools
import math

import jax
import jax.numpy as jnp
from jax import lax
from jax.experimental import pallas as pl
from jax.experimental.pallas import tpu as pltpu
from jax.experimental.pallas import tpu_sc as plsc

D_MODEL = 64
NUM_HEADS = 8
DT = D_MODEL * NUM_HEADS
VOCAB = 130
PAD_IDX = VOCAB // 2
SCALE = math.sqrt(D_MODEL)

B = 512 * 512
NC, NS = 2, 16
NW = NC * NS
BPW = B // NW
K = 64
NSTEP = BPW // K


def _prep_body(tbl_ref, out_ref):
    rows = lax.broadcasted_iota(jnp.int32, (VOCAB, D_MODEL), 0)
    t = jnp.where(rows == PAD_IDX, 0.0, tbl_ref[...]) * SCALE
    out_ref[...] = jnp.concatenate([t] * NUM_HEADS, axis=1)


def _make_tiled_table(tbl):
    return pl.pallas_call(
        _prep_body,
        out_shape=jax.ShapeDtypeStruct((VOCAB, DT), jnp.float32),
    )(tbl)


_sc_mesh = plsc.VectorSubcoreMesh(core_axis_name="c", subcore_axis_name="s")


@functools.partial(
    pl.kernel,
    out_type=jax.ShapeDtypeStruct((B, DT), jnp.float32),
    mesh=_sc_mesh,
    scratch_types=[
        pltpu.VMEM((NSTEP, K), jnp.int32),
        pltpu.VMEM((K, DT), jnp.float32),
        pltpu.VMEM((K, DT), jnp.float32),
        pltpu.SemaphoreType.DMA,
        pltpu.SemaphoreType.DMA,
        pltpu.SemaphoreType.DMA,
        pltpu.SemaphoreType.DMA,
    ],
)
def _sc_lookup(tbl_hbm, idx_hbm, out_hbm, idx_v, buf0, buf1, g0, g1, s0, s1):
    wid = lax.axis_index("s") * NC + lax.axis_index("c")
    base = wid * BPW
    pltpu.sync_copy(idx_hbm.at[wid], idx_v)

    bufs = (buf0, buf1)
    gsems = (g0, g1)
    ssems = (s0, s1)

    def gather_start(j, s):
        pltpu.async_copy(tbl_hbm.at[idx_v.at[j]], bufs[s], gsems[s])

    def gather_wait(j, s):
        pltpu.make_async_copy(tbl_hbm.at[idx_v.at[j]], bufs[s], gsems[s]).wait()

    def put_start(j, s):
        pltpu.async_copy(bufs[s], out_hbm.at[pl.ds(base + j * K, K)], ssems[s])

    def put_wait(j, s):
        pltpu.make_async_copy(
            bufs[s], out_hbm.at[pl.ds(base + j * K, K)], ssems[s]
        ).wait()

    gather_start(0, 0)
    gather_start(1, 1)

    def pair(p, carry):
        j0 = p * 2
        for s in range(2):
            j = j0 + s
            gather_wait(j, s)
            put_start(j, s)

            @pl.when(j + 2 < NSTEP)
            def _():
                put_wait(j, s)
                gather_start(j + 2, s)

        return carry

    lax.fori_loop(0, NSTEP // 2, pair, 0)

    put_wait(NSTEP - 2, 0)
    put_wait(NSTEP - 1, 1)


def kernel(inputs, rel_emb_v_weight):
    tiled = _make_tiled_table(rel_emb_v_weight)
    idx = inputs.reshape(NW, NSTEP, K).astype(jnp.int32)
    out = _sc_lookup(tiled, idx)
    return out.reshape(1, 512, 512, DT)
